# Initial kernel scaffold; baseline (speedup 1.0000x reference)
import jax
import jax.numpy as jnp
from jax.experimental import pallas as pl


def kernel(inputs, adj, W1, b1, W2, b2, W3, b3, W4, b4, W5, b5, W6, b6, W7, b7, W8, b8):
    raise NotImplementedError("write your pallas kernel here")


# speedup vs baseline: 26.6474x; 26.6474x over previous
import functools

import jax
import jax.numpy as jnp
from jax import lax
from jax.experimental import pallas as pl
from jax.experimental.pallas import tpu as pltpu
from jax.experimental.pallas import tpu_sc as plsc

N = 10000
E = 320000
D = 128

NC = 2
NS = 16
NW = NC * NS
CH = 128
CPW = -(-E // (NW * CH))
EPW = CPW * CH
EPAD = NW * EPW
NPAD = 10240
RPT = NPAD // NS
DUMP = N

_mesh = plsc.VectorSubcoreMesh(core_axis_name="c", subcore_axis_name="s")


@functools.partial(
    pl.kernel,
    out_type=jax.ShapeDtypeStruct((NC, NPAD, D), jnp.float32),
    mesh=_mesh,
    scratch_types=[
        pltpu.VMEM((CPW, CH), jnp.int32),
        pltpu.VMEM((CH, D), jnp.float32),
        pltpu.VMEM_SHARED((NPAD, D), jnp.float32),
    ],
)
def _sc_degree(dst_hbm, ones_hbm, zeros_hbm, out_hbm, idx_v, ones_v, acc):
    cid = lax.axis_index("c")
    sid = lax.axis_index("s")
    wid = cid * NS + sid
    base = sid * RPT
    pltpu.sync_copy(zeros_hbm.at[pl.ds(base, RPT)], acc.at[pl.ds(base, RPT)])
    pltpu.sync_copy(ones_hbm, ones_v)
    pltpu.sync_copy(dst_hbm.at[wid], idx_v)
    plsc.subcore_barrier()

    def body(j, carry):
        pltpu.sync_copy(ones_v, acc.at[idx_v.at[j]], add=True)
        return carry

    lax.fori_loop(0, CPW, body, 0)
    plsc.subcore_barrier()
    pltpu.sync_copy(acc.at[pl.ds(base, RPT)], out_hbm.at[cid, pl.ds(base, RPT)])


@functools.partial(
    pl.kernel,
    out_type=jax.ShapeDtypeStruct((NC, NPAD, D), jnp.float32),
    mesh=_mesh,
    scratch_types=[
        pltpu.VMEM((CPW, CH), jnp.int32),
        pltpu.VMEM((CPW, CH), jnp.int32),
        pltpu.VMEM((CH, D), jnp.float32),
        pltpu.VMEM_SHARED((NPAD, D), jnp.float32),
        pltpu.SemaphoreType.DMA,
    ],
)
def _sc_aggregate(g_hbm, src_hbm, dst_hbm, out_hbm, idx_s, idx_d, rows, acc, sem):
    cid = lax.axis_index("c")
    sid = lax.axis_index("s")
    wid = cid * NS + sid
    base = sid * RPT
    pltpu.sync_copy(g_hbm.at[pl.ds(base, RPT)], acc.at[pl.ds(base, RPT)])
    pltpu.sync_copy(src_hbm.at[wid], idx_s)
    pltpu.sync_copy(dst_hbm.at[wid], idx_d)
    plsc.subcore_barrier()

    def body(j, carry):
        pltpu.async_copy(g_hbm.at[idx_s.at[j]], rows, sem).wait()
        pltpu.sync_copy(rows, acc.at[idx_d.at[j]], add=True)
        return carry

    lax.fori_loop(0, CPW, body, 0)
    plsc.subcore_barrier()
    pltpu.sync_copy(acc.at[pl.ds(base, RPT)], out_hbm.at[cid, pl.ds(base, RPT)])


_BR = 2048


def _prep_body(x_ref, ws_ref, dp_ref, g_ref, dinv_ref):
    W = ws_ref[0]
    for i in range(1, 7):
        W = W + ws_ref[i]
    h = jnp.dot(x_ref[...], W, preferred_element_type=jnp.float32)
    deg = dp_ref[0, :, 0:1] + dp_ref[1, :, 0:1] + 1.0
    dinv = lax.rsqrt(deg)
    g_ref[...] = h * dinv
    dinv_ref[...] = dinv


def _mid_body(p_ref, g_ref, dinv_ref, bs_ref, w8_ref, g2_ref):
    bsum = jnp.sum(bs_ref[...], axis=0, keepdims=True)
    dinv = dinv_ref[...]
    s = p_ref[0] + p_ref[1] - g_ref[...]
    x = jnp.maximum(dinv * s + bsum, 0.0)
    h2 = jnp.dot(x, w8_ref[...], preferred_element_type=jnp.float32)
    g2_ref[...] = h2 * dinv


def _final_body(p_ref, g2_ref, dinv_ref, b8_ref, out_ref):
    s = p_ref[0] + p_ref[1] - g2_ref[...]
    out_ref[...] = dinv_ref[...] * s + b8_ref[...]


def _row_spec(width):
    return pl.BlockSpec((_BR, width), lambda i: (i, 0))


def _part_spec(width):
    return pl.BlockSpec((NC, _BR, width), lambda i: (0, i, 0))


def _full_spec(shape):
    nd = len(shape)
    return pl.BlockSpec(shape, lambda i, _nd=nd: (0,) * _nd)


def kernel(inputs, adj, W1, b1, W2, b2, W3, b3, W4, b4, W5, b5, W6, b6, W7, b7, W8, b8):
    f32 = jnp.float32
    grid = NPAD // _BR

    src = jnp.concatenate([adj[0], jnp.zeros((EPAD - E,), jnp.int32)])
    dst = jnp.concatenate([adj[1], jnp.full((EPAD - E,), DUMP, jnp.int32)])
    src = src.reshape(NW, CPW, CH)
    dst = dst.reshape(NW, CPW, CH)
    xpad = jnp.pad(inputs, ((0, NPAD - N), (0, 0)))
    ws = jnp.stack([W1, W2, W3, W4, W5, W6, W7])
    bs = jnp.stack([b1, b2, b3, b4, b5, b6, b7])
    b8r = b8.reshape(1, D)
    ones_rows = jnp.ones((CH, D), f32)
    zeros_rows = jnp.zeros((NPAD, D), f32)

    degp = _sc_degree(dst, ones_rows, zeros_rows)

    g1, dinv = pl.pallas_call(
        _prep_body,
        grid=(grid,),
        in_specs=[_row_spec(D), _full_spec((7, D, D)), _part_spec(D)],
        out_specs=[_row_spec(D), _row_spec(1)],
        out_shape=[
            jax.ShapeDtypeStruct((NPAD, D), f32),
            jax.ShapeDtypeStruct((NPAD, 1), f32),
        ],
    )(xpad, ws, degp)

    p1 = _sc_aggregate(g1, src, dst)

    g2 = pl.pallas_call(
        _mid_body,
        grid=(grid,),
        in_specs=[_part_spec(D), _row_spec(D), _row_spec(1),
                  _full_spec((7, D)), _full_spec((D, D))],
        out_specs=_row_spec(D),
        out_shape=jax.ShapeDtypeStruct((NPAD, D), f32),
    )(p1, g1, dinv, bs, W8)

    p2 = _sc_aggregate(g2, src, dst)

    out = pl.pallas_call(
        _final_body,
        grid=(grid,),
        in_specs=[_part_spec(D), _row_spec(D), _row_spec(1), _full_spec((1, D))],
        out_specs=_row_spec(D),
        out_shape=jax.ShapeDtypeStruct((NPAD, D), f32),
    )(p2, g2, dinv, b8r)

    return out[:N]
